# initial kernel scaffold (unmeasured)
import jax
import jax.numpy as jnp
from jax import lax
from jax.experimental import pallas as pl
from jax.experimental.pallas import tpu as pltpu


def kernel(
    x,
):
    def body(*refs):
        pass

    out_shape = jax.ShapeDtypeStruct(..., jnp.float32)
    return pl.pallas_call(body, out_shape=out_shape)(...)


# baseline (device time: 7131 ns/iter reference)
import jax
import jax.numpy as jnp
from jax import lax
from jax.experimental import pallas as pl
from jax.experimental.pallas import tpu as pltpu


def kernel(x):
    m, n = x.shape

    def body(x_ref, o_ref, row_send, row_recv, col_send, col_recv,
             send_sems, recv_sems):
        my_x = lax.axis_index("x")
        my_y = lax.axis_index("y")
        x_nbr = (1 - my_x, my_y)
        y_nbr = (my_x, 1 - my_y)

        barrier_sem = pltpu.get_barrier_semaphore()
        for nbr in (x_nbr, y_nbr):
            pl.semaphore_signal(
                barrier_sem, inc=1,
                device_id=nbr, device_id_type=pl.DeviceIdType.MESH,
            )
        pl.semaphore_wait(barrier_sem, 2)

        @pl.when(my_x == 0)
        def _():
            row_send[0, :] = x_ref[m - 1, :]

        @pl.when(my_x == 1)
        def _():
            row_send[0, :] = x_ref[0, :]

        @pl.when(my_y == 0)
        def _():
            col_send[:, 0] = x_ref[:, n - 1]

        @pl.when(my_y == 1)
        def _():
            col_send[:, 0] = x_ref[:, 0]

        row_rdma = pltpu.make_async_remote_copy(
            src_ref=row_send, dst_ref=row_recv,
            send_sem=send_sems.at[0], recv_sem=recv_sems.at[0],
            device_id=x_nbr, device_id_type=pl.DeviceIdType.MESH,
        )
        col_rdma = pltpu.make_async_remote_copy(
            src_ref=col_send, dst_ref=col_recv,
            send_sem=send_sems.at[1], recv_sem=recv_sems.at[1],
            device_id=y_nbr, device_id_type=pl.DeviceIdType.MESH,
        )
        row_rdma.start()
        col_rdma.start()
        row_rdma.wait()
        col_rdma.wait()

        xv = x_ref[:, :]
        rr = row_recv[:, :]
        cr = col_recv[:, :]

        zero_row = jnp.zeros_like(rr)
        zero_col = jnp.zeros_like(cr)
        north_halo = jnp.where(my_x == 1, rr, zero_row)
        south_halo = jnp.where(my_x == 0, rr, zero_row)
        west_halo = jnp.where(my_y == 1, cr, zero_col)
        east_halo = jnp.where(my_y == 0, cr, zero_col)

        north = jnp.concatenate([north_halo, xv[:-1, :]], axis=0)
        south = jnp.concatenate([xv[1:, :], south_halo], axis=0)
        west = jnp.concatenate([west_halo, xv[:, :-1]], axis=1)
        east = jnp.concatenate([xv[:, 1:], east_halo], axis=1)

        stencil = 0.5 * xv + 0.125 * (north + south + east + west)

        grow = lax.broadcasted_iota(jnp.int32, (m, n), 0) + my_x * m
        gcol = lax.broadcasted_iota(jnp.int32, (m, n), 1) + my_y * n
        boundary = (
            (grow == 0) | (grow == 2 * m - 1) | (gcol == 0) | (gcol == 2 * n - 1)
        )
        o_ref[:, :] = jnp.where(boundary, xv, stencil)

    return pl.pallas_call(
        body,
        out_shape=jax.ShapeDtypeStruct((m, n), x.dtype),
        in_specs=[pl.BlockSpec(memory_space=pltpu.VMEM)],
        out_specs=pl.BlockSpec(memory_space=pltpu.VMEM),
        scratch_shapes=[
            pltpu.VMEM((1, n), x.dtype),
            pltpu.VMEM((1, n), x.dtype),
            pltpu.VMEM((m, 1), x.dtype),
            pltpu.VMEM((m, 1), x.dtype),
            pltpu.SemaphoreType.DMA((2,)),
            pltpu.SemaphoreType.DMA((2,)),
        ],
        compiler_params=pltpu.CompilerParams(collective_id=0),
    )(x)
